# Initial kernel scaffold; baseline (speedup 1.0000x reference)
import jax
import jax.numpy as jnp
from jax.experimental import pallas as pl


def kernel(data, atom_table, params, Wc, bc):
    raise NotImplementedError("write your pallas kernel here")


# speedup vs baseline: 5.1968x; 5.1968x over previous
import functools
import numpy as np
import jax
import jax.numpy as jnp
from jax import lax
from jax.experimental import pallas as pl
from jax.experimental.pallas import tpu as pltpu
from jax.experimental.pallas import tpu_sc as plsc

_FILT = np.arange(0.0, 8.5, 0.5, dtype=np.float32)
_HA = 16
_TN = 200
_CHUNK = 128


def _sc_gather(table, idx2d):
    nrow = idx2d.shape[0]
    rows_total = nrow * _CHUNK
    info = plsc.get_sparse_core_info()
    nw = info.num_cores * info.num_subcores
    rpw = nrow // nw

    idx3 = idx2d.reshape(nw, rpw, _CHUNK)
    mesh = plsc.VectorSubcoreMesh(core_axis_name="c", subcore_axis_name="s")

    @functools.partial(
        pl.kernel,
        mesh=mesh,
        compiler_params=pltpu.CompilerParams(use_tc_tiling_on_sc=False),
        out_type=jax.ShapeDtypeStruct((rows_total, _HA), jnp.float32),
        scratch_types=[
            pltpu.VMEM((rpw, _CHUNK), jnp.int32),
            pltpu.VMEM((_CHUNK, _HA), jnp.float32),
            pltpu.SemaphoreType.DMA,
        ],
    )
    def k(table_hbm, idx_hbm, out_hbm, idx_v, rows_v, sem):
        wid = lax.axis_index("s") * info.num_cores + lax.axis_index("c")
        base = wid * rpw
        pltpu.sync_copy(idx_hbm.at[wid], idx_v)

        def body(i, _):
            pltpu.async_copy(table_hbm.at[idx_v.at[i]], rows_v, sem).wait()
            pltpu.sync_copy(rows_v, out_hbm.at[pl.ds((base + i) * _CHUNK, _CHUNK)])
            return _

        lax.fori_loop(0, rpw, body, None)

    return k(table, idx3)


def _gated_halves(dist_ref, gath_ref, atom_ref, e_ref, filt_ref,
                  w1f, w1c, w2f, w2c, w3f, w3c, bf, bc_):
    d = dist_ref[0]
    nbr = jnp.exp(-4.0 * (d - filt_ref[...]) ** 2)
    g16 = gath_ref[...]
    a16 = atom_ref[0]
    e = e_ref[...]
    gf = (jnp.dot(nbr, w3f[...], preferred_element_type=jnp.float32)
          + jnp.dot(g16, w2f[...], preferred_element_type=jnp.float32)
          + jnp.dot(e, jnp.dot(a16, w1f[...], preferred_element_type=jnp.float32),
                    preferred_element_type=jnp.float32)
          + bf[...])
    gc = (jnp.dot(nbr, w3c[...], preferred_element_type=jnp.float32)
          + jnp.dot(g16, w2c[...], preferred_element_type=jnp.float32)
          + jnp.dot(e, jnp.dot(a16, w1c[...], preferred_element_type=jnp.float32),
                    preferred_element_type=jnp.float32)
          + bc_[...])
    return gf, gc


def _pass1_body(dist_ref, gath_ref, atom_ref, e_ref, filt_ref,
                w1f, w1c, w2f, w2c, w3f, w3c, bf, bc_, stats_ref):
    gf, gc = _gated_halves(dist_ref, gath_ref, atom_ref, e_ref, filt_ref,
                           w1f, w1c, w2f, w2c, w3f, w3c, bf, bc_)
    st = jnp.concatenate(
        [jnp.sum(gf, 0, keepdims=True), jnp.sum(gf * gf, 0, keepdims=True),
         jnp.sum(gc, 0, keepdims=True), jnp.sum(gc * gc, 0, keepdims=True)], axis=0)

    @pl.when((pl.program_id(0) == 0) & (pl.program_id(1) == 0))
    def _():
        stats_ref[...] = jnp.zeros_like(stats_ref)

    stats_ref[...] += st


def _pass2_body(dist_ref, gath_ref, atom_ref, e_ref, et_ref, filt_ref,
                w1f, w1c, w2f, w2c, w3f, w3c, bf, bc_,
                af_ref, cf_ref, ac_ref, cc_ref, sum_ref, stats_ref):
    gf, gc = _gated_halves(dist_ref, gath_ref, atom_ref, e_ref, filt_ref,
                           w1f, w1c, w2f, w2c, w3f, w3c, bf, bc_)
    filt = jax.nn.sigmoid(gf * af_ref[...] + cf_ref[...])
    core = jnp.maximum(gc * ac_ref[...] + cc_ref[...], 0.0)
    prod = filt * core
    sm = jnp.dot(et_ref[...], prod, preferred_element_type=jnp.float32)
    sum_ref[...] = sm[None]
    st = jnp.concatenate(
        [jnp.sum(sm, 0, keepdims=True), jnp.sum(sm * sm, 0, keepdims=True)], axis=0)

    @pl.when((pl.program_id(0) == 0) & (pl.program_id(1) == 0))
    def _():
        stats_ref[...] = jnp.zeros_like(stats_ref)

    stats_ref[...] += st


def _pass3_body(sum_ref, atom_ref, a2_ref, c2_ref, out_ref):
    out_ref[...] = jnp.maximum(
        atom_ref[...] + sum_ref[...] * a2_ref[...] + c2_ref[...], 0.0)


def _head_body(atom_ref, wc_ref, bcl_ref, out_ref):
    rows = [jnp.sum(atom_ref[b], axis=0, keepdims=True) for b in range(atom_ref.shape[0])]
    pooled = jnp.concatenate(rows, axis=0) * (1.0 / atom_ref.shape[1])
    logits = jnp.dot(pooled, wc_ref[...], preferred_element_type=jnp.float32) + bcl_ref[...]
    m = jnp.max(logits, axis=-1, keepdims=True)
    ex = jnp.exp(logits - m)
    out_ref[...] = ex / jnp.sum(ex, axis=-1, keepdims=True)


def kernel(data, atom_table, params, Wc, bc):
    B, N, n2 = data.shape
    M = n2 // 2
    T = _TN * M
    nb = N // _TN
    R = B * N * M

    dist = data[:, :, :M]
    nbr_idx = data[:, :, M:].astype(jnp.int32)

    dist_c = dist.reshape(B, N * M, 1)
    idx_flat = nbr_idx.reshape(B, N * M)
    idx_l1 = idx_flat.reshape(R // _CHUNK, _CHUNK)
    offs = (jnp.arange(B, dtype=jnp.int32) * N)[:, None]
    idx_l2 = (idx_flat + offs).reshape(R // _CHUNK, _CHUNK)

    e_mat = jnp.asarray(np.repeat(np.eye(_TN, dtype=np.float32), M, axis=0))
    et_mat = e_mat.T
    filt_arr = jnp.asarray(_FILT)[None]

    emb = jnp.broadcast_to(atom_table[None], (B, N, _HA))

    grid = (B, nb)
    full = lambda shape: pl.BlockSpec(shape, lambda b, i: (0,) * len(shape))
    gath_spec = pl.BlockSpec((T, _HA), lambda b, i: (b * nb + i, 0))
    dist_spec = pl.BlockSpec((1, T, 1), lambda b, i: (b, i, 0))
    atom_spec = pl.BlockSpec((1, _TN, _HA), lambda b, i: (b, i, 0))

    for layer, p in enumerate(params):
        W = p['W']
        w1f, w1c = W[:_HA, :_HA], W[:_HA, _HA:]
        w2f, w2c = W[_HA:2 * _HA, :_HA], W[_HA:2 * _HA, _HA:]
        w3f, w3c = W[2 * _HA:, :_HA], W[2 * _HA:, _HA:]
        bf, bc_ = p['b'][None, :_HA], p['b'][None, _HA:]

        if layer == 0:
            gath = _sc_gather(atom_table, idx_l1)
        else:
            gath = _sc_gather(emb.reshape(B * N, _HA), idx_l2)

        win = [full(w1f.shape), full(w1c.shape), full(w2f.shape), full(w2c.shape),
               full(w3f.shape), full(w3c.shape), full(bf.shape), full(bc_.shape)]

        stats1 = pl.pallas_call(
            _pass1_body,
            grid=grid,
            in_specs=[dist_spec, gath_spec, atom_spec, full(e_mat.shape),
                      full(filt_arr.shape)] + win,
            out_specs=full((4, _HA)),
            out_shape=jax.ShapeDtypeStruct((4, _HA), jnp.float32),
        )(dist_c, gath, emb, e_mat, filt_arr, w1f, w1c, w2f, w2c, w3f, w3c, bf, bc_)

        cnt1 = float(B * N * M)
        muf, mu2f = stats1[0] / cnt1, stats1[1] / cnt1
        muc, mu2c = stats1[2] / cnt1, stats1[3] / cnt1
        invf = jax.lax.rsqrt(mu2f - muf * muf + 1e-5)
        invc = jax.lax.rsqrt(mu2c - muc * muc + 1e-5)
        af = (p['gh'][:_HA] * invf)[None]
        cf = (p['bh'][:_HA] - muf * p['gh'][:_HA] * invf)[None]
        ac = (p['gh'][_HA:] * invc)[None]
        cc = (p['bh'][_HA:] - muc * p['gh'][_HA:] * invc)[None]

        summed, stats2 = pl.pallas_call(
            _pass2_body,
            grid=grid,
            in_specs=[dist_spec, gath_spec, atom_spec, full(e_mat.shape),
                      full(et_mat.shape), full(filt_arr.shape)] + win +
                     [full((1, _HA)), full((1, _HA)), full((1, _HA)), full((1, _HA))],
            out_specs=[atom_spec, full((2, _HA))],
            out_shape=[jax.ShapeDtypeStruct((B, N, _HA), jnp.float32),
                       jax.ShapeDtypeStruct((2, _HA), jnp.float32)],
        )(dist_c, gath, emb, e_mat, et_mat, filt_arr, w1f, w1c, w2f, w2c, w3f, w3c,
          bf, bc_, af, cf, ac, cc)

        cnt2 = float(B * N)
        mu2 = stats2[0] / cnt2
        var2 = stats2[1] / cnt2 - mu2 * mu2
        inv2 = jax.lax.rsqrt(var2 + 1e-5)
        a2 = (p['go'] * inv2)[None, None]
        c2 = (p['bo'] - mu2 * p['go'] * inv2)[None, None]

        emb = pl.pallas_call(
            _pass3_body,
            grid=(B,),
            in_specs=[pl.BlockSpec((1, N, _HA), lambda b: (b, 0, 0)),
                      pl.BlockSpec((1, N, _HA), lambda b: (b, 0, 0)),
                      pl.BlockSpec((1, 1, _HA), lambda b: (0, 0, 0)),
                      pl.BlockSpec((1, 1, _HA), lambda b: (0, 0, 0))],
            out_specs=pl.BlockSpec((1, N, _HA), lambda b: (b, 0, 0)),
            out_shape=jax.ShapeDtypeStruct((B, N, _HA), jnp.float32),
        )(summed, emb, a2, c2)

    out = pl.pallas_call(
        _head_body,
        in_specs=[pl.BlockSpec(emb.shape, lambda: (0, 0, 0)),
                  pl.BlockSpec(Wc.shape, lambda: (0, 0)),
                  pl.BlockSpec((1, Wc.shape[1]), lambda: (0, 0))],
        out_specs=pl.BlockSpec((B, Wc.shape[1]), lambda: (0, 0)),
        out_shape=jax.ShapeDtypeStruct((B, Wc.shape[1]), jnp.float32),
    )(emb, Wc, bc[None])
    return out


# speedup vs baseline: 5.3580x; 1.0310x over previous
import functools
import numpy as np
import jax
import jax.numpy as jnp
from jax import lax
from jax.experimental import pallas as pl
from jax.experimental.pallas import tpu as pltpu
from jax.experimental.pallas import tpu_sc as plsc

_FILT = np.arange(0.0, 8.5, 0.5, dtype=np.float32)
_HA = 16
_TN = 200
_CHUNK = 128
_NBUF = 5


def _sc_gather(table, idx2d):
    nrow = idx2d.shape[0]
    rows_total = nrow * _CHUNK
    info = plsc.get_sparse_core_info()
    nw = info.num_cores * info.num_subcores
    rpw = nrow // nw

    idx3 = idx2d.reshape(nw, rpw, _CHUNK)
    mesh = plsc.VectorSubcoreMesh(core_axis_name="c", subcore_axis_name="s")

    @functools.partial(
        pl.kernel,
        mesh=mesh,
        compiler_params=pltpu.CompilerParams(use_tc_tiling_on_sc=False),
        out_type=jax.ShapeDtypeStruct((rows_total, _HA), jnp.float32),
        scratch_types=[
            pltpu.VMEM((rpw, _CHUNK), jnp.int32),
            pltpu.VMEM((_NBUF, _CHUNK, _HA), jnp.float32),
            pltpu.SemaphoreType.DMA,
            pltpu.SemaphoreType.DMA,
        ],
    )
    def k(table_hbm, idx_hbm, out_hbm, idx_v, rows_v, sem_g, sem_w):
        wid = lax.axis_index("s") * info.num_cores + lax.axis_index("c")
        base = wid * rpw

        pltpu.sync_copy(idx_hbm.at[wid], idx_v)

        def body(j, _):
            @pl.when(j > 0)
            def _():
                for p in range(_NBUF):
                    pltpu.make_async_copy(
                        rows_v.at[p],
                        out_hbm.at[pl.ds(base * _CHUNK, _CHUNK)], sem_w).wait()

            gs = [pltpu.async_copy(table_hbm.at[idx_v.at[j * _NBUF + p]],
                                   rows_v.at[p], sem_g)
                  for p in range(_NBUF)]
            for p in range(_NBUF):
                gs[p].wait()
                pltpu.async_copy(
                    rows_v.at[p],
                    out_hbm.at[pl.ds((base + j * _NBUF + p) * _CHUNK, _CHUNK)],
                    sem_w)
            return _

        lax.fori_loop(0, rpw // _NBUF, body, None)
        for p in range(_NBUF):
            pltpu.make_async_copy(
                rows_v.at[p], out_hbm.at[pl.ds(base * _CHUNK, _CHUNK)], sem_w).wait()

    return k(table, idx3)


def _gated_halves(dist_ref, gath_ref, atom_ref, e_ref, filt_ref,
                  w1f, w1c, w2f, w2c, w3f, w3c, bf, bc_):
    d = dist_ref[0]
    nbr = jnp.exp(-4.0 * (d - filt_ref[...]) ** 2)
    g16 = gath_ref[...]
    a16 = atom_ref[0]
    e = e_ref[...]
    gf = (jnp.dot(nbr, w3f[...], preferred_element_type=jnp.float32)
          + jnp.dot(g16, w2f[...], preferred_element_type=jnp.float32)
          + jnp.dot(e, jnp.dot(a16, w1f[...], preferred_element_type=jnp.float32),
                    preferred_element_type=jnp.float32)
          + bf[...])
    gc = (jnp.dot(nbr, w3c[...], preferred_element_type=jnp.float32)
          + jnp.dot(g16, w2c[...], preferred_element_type=jnp.float32)
          + jnp.dot(e, jnp.dot(a16, w1c[...], preferred_element_type=jnp.float32),
                    preferred_element_type=jnp.float32)
          + bc_[...])
    return gf, gc


def _pass1_body(dist_ref, gath_ref, atom_ref, e_ref, filt_ref,
                w1f, w1c, w2f, w2c, w3f, w3c, bf, bc_, stats_ref):
    gf, gc = _gated_halves(dist_ref, gath_ref, atom_ref, e_ref, filt_ref,
                           w1f, w1c, w2f, w2c, w3f, w3c, bf, bc_)
    st = jnp.concatenate(
        [jnp.sum(gf, 0, keepdims=True), jnp.sum(gf * gf, 0, keepdims=True),
         jnp.sum(gc, 0, keepdims=True), jnp.sum(gc * gc, 0, keepdims=True)], axis=0)

    @pl.when((pl.program_id(0) == 0) & (pl.program_id(1) == 0))
    def _():
        stats_ref[...] = jnp.zeros_like(stats_ref)

    stats_ref[...] += st


def _pass2_body(dist_ref, gath_ref, atom_ref, e_ref, et_ref, filt_ref,
                w1f, w1c, w2f, w2c, w3f, w3c, bf, bc_,
                af_ref, cf_ref, ac_ref, cc_ref, sum_ref, stats_ref):
    gf, gc = _gated_halves(dist_ref, gath_ref, atom_ref, e_ref, filt_ref,
                           w1f, w1c, w2f, w2c, w3f, w3c, bf, bc_)
    filt = jax.nn.sigmoid(gf * af_ref[...] + cf_ref[...])
    core = jnp.maximum(gc * ac_ref[...] + cc_ref[...], 0.0)
    prod = filt * core
    sm = jnp.dot(et_ref[...], prod, preferred_element_type=jnp.float32)
    sum_ref[...] = sm[None]
    st = jnp.concatenate(
        [jnp.sum(sm, 0, keepdims=True), jnp.sum(sm * sm, 0, keepdims=True)], axis=0)

    @pl.when((pl.program_id(0) == 0) & (pl.program_id(1) == 0))
    def _():
        stats_ref[...] = jnp.zeros_like(stats_ref)

    stats_ref[...] += st


def _pass3_body(sum_ref, atom_ref, a2_ref, c2_ref, out_ref):
    out_ref[...] = jnp.maximum(
        atom_ref[...] + sum_ref[...] * a2_ref[...] + c2_ref[...], 0.0)


def _head_body(atom_ref, wc_ref, bcl_ref, out_ref):
    rows = [jnp.sum(atom_ref[b], axis=0, keepdims=True) for b in range(atom_ref.shape[0])]
    pooled = jnp.concatenate(rows, axis=0) * (1.0 / atom_ref.shape[1])
    logits = jnp.dot(pooled, wc_ref[...], preferred_element_type=jnp.float32) + bcl_ref[...]
    m = jnp.max(logits, axis=-1, keepdims=True)
    ex = jnp.exp(logits - m)
    out_ref[...] = ex / jnp.sum(ex, axis=-1, keepdims=True)


def kernel(data, atom_table, params, Wc, bc):
    B, N, n2 = data.shape
    M = n2 // 2
    T = _TN * M
    nb = N // _TN
    R = B * N * M

    dist = data[:, :, :M]
    nbr_idx = data[:, :, M:].astype(jnp.int32)

    dist_c = dist.reshape(B, N * M, 1)
    idx_flat = nbr_idx.reshape(B, N * M)
    idx_l1 = idx_flat.reshape(R // _CHUNK, _CHUNK)
    offs = (jnp.arange(B, dtype=jnp.int32) * N)[:, None]
    idx_l2 = (idx_flat + offs).reshape(R // _CHUNK, _CHUNK)

    e_mat = jnp.asarray(np.repeat(np.eye(_TN, dtype=np.float32), M, axis=0))
    et_mat = e_mat.T
    filt_arr = jnp.asarray(_FILT)[None]

    emb = jnp.broadcast_to(atom_table[None], (B, N, _HA))

    grid = (B, nb)
    full = lambda shape: pl.BlockSpec(shape, lambda b, i: (0,) * len(shape))
    gath_spec = pl.BlockSpec((T, _HA), lambda b, i: (b * nb + i, 0))
    dist_spec = pl.BlockSpec((1, T, 1), lambda b, i: (b, i, 0))
    atom_spec = pl.BlockSpec((1, _TN, _HA), lambda b, i: (b, i, 0))

    for layer, p in enumerate(params):
        W = p['W']
        w1f, w1c = W[:_HA, :_HA], W[:_HA, _HA:]
        w2f, w2c = W[_HA:2 * _HA, :_HA], W[_HA:2 * _HA, _HA:]
        w3f, w3c = W[2 * _HA:, :_HA], W[2 * _HA:, _HA:]
        bf, bc_ = p['b'][None, :_HA], p['b'][None, _HA:]

        if layer == 0:
            gath = _sc_gather(atom_table, idx_l1)
        else:
            gath = _sc_gather(emb.reshape(B * N, _HA), idx_l2)

        win = [full(w1f.shape), full(w1c.shape), full(w2f.shape), full(w2c.shape),
               full(w3f.shape), full(w3c.shape), full(bf.shape), full(bc_.shape)]

        stats1 = pl.pallas_call(
            _pass1_body,
            grid=grid,
            in_specs=[dist_spec, gath_spec, atom_spec, full(e_mat.shape),
                      full(filt_arr.shape)] + win,
            out_specs=full((4, _HA)),
            out_shape=jax.ShapeDtypeStruct((4, _HA), jnp.float32),
        )(dist_c, gath, emb, e_mat, filt_arr, w1f, w1c, w2f, w2c, w3f, w3c, bf, bc_)

        cnt1 = float(B * N * M)
        muf, mu2f = stats1[0] / cnt1, stats1[1] / cnt1
        muc, mu2c = stats1[2] / cnt1, stats1[3] / cnt1
        invf = jax.lax.rsqrt(mu2f - muf * muf + 1e-5)
        invc = jax.lax.rsqrt(mu2c - muc * muc + 1e-5)
        af = (p['gh'][:_HA] * invf)[None]
        cf = (p['bh'][:_HA] - muf * p['gh'][:_HA] * invf)[None]
        ac = (p['gh'][_HA:] * invc)[None]
        cc = (p['bh'][_HA:] - muc * p['gh'][_HA:] * invc)[None]

        summed, stats2 = pl.pallas_call(
            _pass2_body,
            grid=grid,
            in_specs=[dist_spec, gath_spec, atom_spec, full(e_mat.shape),
                      full(et_mat.shape), full(filt_arr.shape)] + win +
                     [full((1, _HA)), full((1, _HA)), full((1, _HA)), full((1, _HA))],
            out_specs=[atom_spec, full((2, _HA))],
            out_shape=[jax.ShapeDtypeStruct((B, N, _HA), jnp.float32),
                       jax.ShapeDtypeStruct((2, _HA), jnp.float32)],
        )(dist_c, gath, emb, e_mat, et_mat, filt_arr, w1f, w1c, w2f, w2c, w3f, w3c,
          bf, bc_, af, cf, ac, cc)

        cnt2 = float(B * N)
        mu2 = stats2[0] / cnt2
        var2 = stats2[1] / cnt2 - mu2 * mu2
        inv2 = jax.lax.rsqrt(var2 + 1e-5)
        a2 = (p['go'] * inv2)[None, None]
        c2 = (p['bo'] - mu2 * p['go'] * inv2)[None, None]

        emb = pl.pallas_call(
            _pass3_body,
            grid=(B,),
            in_specs=[pl.BlockSpec((1, N, _HA), lambda b: (b, 0, 0)),
                      pl.BlockSpec((1, N, _HA), lambda b: (b, 0, 0)),
                      pl.BlockSpec((1, 1, _HA), lambda b: (0, 0, 0)),
                      pl.BlockSpec((1, 1, _HA), lambda b: (0, 0, 0))],
            out_specs=pl.BlockSpec((1, N, _HA), lambda b: (b, 0, 0)),
            out_shape=jax.ShapeDtypeStruct((B, N, _HA), jnp.float32),
        )(summed, emb, a2, c2)

    out = pl.pallas_call(
        _head_body,
        in_specs=[pl.BlockSpec(emb.shape, lambda: (0, 0, 0)),
                  pl.BlockSpec(Wc.shape, lambda: (0, 0)),
                  pl.BlockSpec((1, Wc.shape[1]), lambda: (0, 0))],
        out_specs=pl.BlockSpec((B, Wc.shape[1]), lambda: (0, 0)),
        out_shape=jax.ShapeDtypeStruct((B, Wc.shape[1]), jnp.float32),
    )(emb, Wc, bc[None])
    return out


# speedup vs baseline: 6.3807x; 1.1909x over previous
import functools
import numpy as np
import jax
import jax.numpy as jnp
from jax import lax
from jax.experimental import pallas as pl
from jax.experimental.pallas import tpu as pltpu
from jax.experimental.pallas import tpu_sc as plsc

_FILT = np.arange(0.0, 8.5, 0.5, dtype=np.float32)
_HA = 16
_TN = 200
_CHUNK = 128
_NBUF = 5


def _sc_gather(table, idx2d):
    nrow = idx2d.shape[0]
    rows_total = nrow * _CHUNK
    info = plsc.get_sparse_core_info()
    nw = info.num_cores * info.num_subcores
    rpw = nrow // nw

    idx3 = idx2d.reshape(nw, rpw, _CHUNK)
    mesh = plsc.VectorSubcoreMesh(core_axis_name="c", subcore_axis_name="s")

    @functools.partial(
        pl.kernel,
        mesh=mesh,
        compiler_params=pltpu.CompilerParams(use_tc_tiling_on_sc=False),
        out_type=jax.ShapeDtypeStruct((rows_total, _HA), jnp.float32),
        scratch_types=[
            pltpu.VMEM((rpw, _CHUNK), jnp.int32),
            pltpu.VMEM((_NBUF, _CHUNK, _HA), jnp.float32),
            pltpu.SemaphoreType.DMA,
            pltpu.SemaphoreType.DMA,
        ],
    )
    def k(table_hbm, idx_hbm, out_hbm, idx_v, rows_v, sem_g, sem_w):
        wid = lax.axis_index("s") * info.num_cores + lax.axis_index("c")
        base = wid * rpw

        pltpu.sync_copy(idx_hbm.at[wid], idx_v)

        def body(j, _):
            @pl.when(j > 0)
            def _():
                for p in range(_NBUF):
                    pltpu.make_async_copy(
                        rows_v.at[p],
                        out_hbm.at[pl.ds(base * _CHUNK, _CHUNK)], sem_w).wait()

            gs = [pltpu.async_copy(table_hbm.at[idx_v.at[j * _NBUF + p]],
                                   rows_v.at[p], sem_g)
                  for p in range(_NBUF)]
            for p in range(_NBUF):
                gs[p].wait()
                pltpu.async_copy(
                    rows_v.at[p],
                    out_hbm.at[pl.ds((base + j * _NBUF + p) * _CHUNK, _CHUNK)],
                    sem_w)
            return _

        lax.fori_loop(0, rpw // _NBUF, body, None)
        for p in range(_NBUF):
            pltpu.make_async_copy(
                rows_v.at[p], out_hbm.at[pl.ds(base * _CHUNK, _CHUNK)], sem_w).wait()

    return k(table, idx3)


def _expand(s, m, t):
    tn, c = s.shape
    return jnp.broadcast_to(s[:, None, :], (tn, m, c)).reshape(t, c)


def _gated_halves(dist_ref, gath_ref, atom_ref, filt_ref,
                  w1f, w1c, w2f, w2c, w3f, w3c, bf, bc_):
    d = dist_ref[0]
    t = d.shape[0]
    nbr = jnp.exp(-4.0 * (d - filt_ref[...]) ** 2)
    g16 = gath_ref[...]
    a16 = atom_ref[0]
    m = t // a16.shape[0]
    gf = (jnp.dot(nbr, w3f[...], preferred_element_type=jnp.float32)
          + jnp.dot(g16, w2f[...], preferred_element_type=jnp.float32)
          + _expand(jnp.dot(a16, w1f[...], preferred_element_type=jnp.float32), m, t)
          + bf[...])
    gc = (jnp.dot(nbr, w3c[...], preferred_element_type=jnp.float32)
          + jnp.dot(g16, w2c[...], preferred_element_type=jnp.float32)
          + _expand(jnp.dot(a16, w1c[...], preferred_element_type=jnp.float32), m, t)
          + bc_[...])
    return gf, gc


def _pass1_body(dist_ref, gath_ref, atom_ref, filt_ref,
                w1f, w1c, w2f, w2c, w3f, w3c, bf, bc_, stats_ref):
    gf, gc = _gated_halves(dist_ref, gath_ref, atom_ref, filt_ref,
                           w1f, w1c, w2f, w2c, w3f, w3c, bf, bc_)
    st = jnp.concatenate(
        [jnp.sum(gf, 0, keepdims=True), jnp.sum(gf * gf, 0, keepdims=True),
         jnp.sum(gc, 0, keepdims=True), jnp.sum(gc * gc, 0, keepdims=True)], axis=0)

    @pl.when((pl.program_id(0) == 0) & (pl.program_id(1) == 0))
    def _():
        stats_ref[...] = jnp.zeros_like(stats_ref)

    stats_ref[...] += st


def _pass2_body(dist_ref, gath_ref, atom_ref, filt_ref,
                w1f, w1c, w2f, w2c, w3f, w3c, bf, bc_,
                af_ref, cf_ref, ac_ref, cc_ref, sum_ref, stats_ref):
    gf, gc = _gated_halves(dist_ref, gath_ref, atom_ref, filt_ref,
                           w1f, w1c, w2f, w2c, w3f, w3c, bf, bc_)
    filt = jax.nn.sigmoid(gf * af_ref[...] + cf_ref[...])
    core = jnp.maximum(gc * ac_ref[...] + cc_ref[...], 0.0)
    prod = filt * core
    tn = atom_ref.shape[1]
    sm = jnp.sum(prod.reshape(tn, prod.shape[0] // tn, prod.shape[1]), axis=1)
    sum_ref[...] = sm[None]
    st = jnp.concatenate(
        [jnp.sum(sm, 0, keepdims=True), jnp.sum(sm * sm, 0, keepdims=True)], axis=0)

    @pl.when((pl.program_id(0) == 0) & (pl.program_id(1) == 0))
    def _():
        stats_ref[...] = jnp.zeros_like(stats_ref)

    stats_ref[...] += st


def _pass3_body(sum_ref, atom_ref, a2_ref, c2_ref, out_ref):
    out_ref[...] = jnp.maximum(
        atom_ref[...] + sum_ref[...] * a2_ref[...] + c2_ref[...], 0.0)


def _head_body(atom_ref, wc_ref, bcl_ref, out_ref):
    rows = [jnp.sum(atom_ref[b], axis=0, keepdims=True) for b in range(atom_ref.shape[0])]
    pooled = jnp.concatenate(rows, axis=0) * (1.0 / atom_ref.shape[1])
    logits = jnp.dot(pooled, wc_ref[...], preferred_element_type=jnp.float32) + bcl_ref[...]
    m = jnp.max(logits, axis=-1, keepdims=True)
    ex = jnp.exp(logits - m)
    out_ref[...] = ex / jnp.sum(ex, axis=-1, keepdims=True)


def kernel(data, atom_table, params, Wc, bc):
    B, N, n2 = data.shape
    M = n2 // 2
    T = _TN * M
    nb = N // _TN
    R = B * N * M

    dist = data[:, :, :M]
    nbr_idx = data[:, :, M:].astype(jnp.int32)

    dist_c = dist.reshape(B, N * M, 1)
    idx_flat = nbr_idx.reshape(B, N * M)
    idx_l1 = idx_flat.reshape(R // _CHUNK, _CHUNK)
    offs = (jnp.arange(B, dtype=jnp.int32) * N)[:, None]
    idx_l2 = (idx_flat + offs).reshape(R // _CHUNK, _CHUNK)

    filt_arr = jnp.asarray(_FILT)[None]

    emb = jnp.broadcast_to(atom_table[None], (B, N, _HA))

    grid = (B, nb)
    full = lambda shape: pl.BlockSpec(shape, lambda b, i: (0,) * len(shape))
    gath_spec = pl.BlockSpec((T, _HA), lambda b, i: (b * nb + i, 0))
    dist_spec = pl.BlockSpec((1, T, 1), lambda b, i: (b, i, 0))
    atom_spec = pl.BlockSpec((1, _TN, _HA), lambda b, i: (b, i, 0))

    for layer, p in enumerate(params):
        W = p['W']
        w1f, w1c = W[:_HA, :_HA], W[:_HA, _HA:]
        w2f, w2c = W[_HA:2 * _HA, :_HA], W[_HA:2 * _HA, _HA:]
        w3f, w3c = W[2 * _HA:, :_HA], W[2 * _HA:, _HA:]
        bf, bc_ = p['b'][None, :_HA], p['b'][None, _HA:]

        if layer == 0:
            gath = _sc_gather(atom_table, idx_l1)
        else:
            gath = _sc_gather(emb.reshape(B * N, _HA), idx_l2)

        win = [full(w1f.shape), full(w1c.shape), full(w2f.shape), full(w2c.shape),
               full(w3f.shape), full(w3c.shape), full(bf.shape), full(bc_.shape)]

        stats1 = pl.pallas_call(
            _pass1_body,
            grid=grid,
            in_specs=[dist_spec, gath_spec, atom_spec, full(filt_arr.shape)] + win,
            out_specs=full((4, _HA)),
            out_shape=jax.ShapeDtypeStruct((4, _HA), jnp.float32),
        )(dist_c, gath, emb, filt_arr, w1f, w1c, w2f, w2c, w3f, w3c, bf, bc_)

        cnt1 = float(B * N * M)
        muf, mu2f = stats1[0] / cnt1, stats1[1] / cnt1
        muc, mu2c = stats1[2] / cnt1, stats1[3] / cnt1
        invf = jax.lax.rsqrt(mu2f - muf * muf + 1e-5)
        invc = jax.lax.rsqrt(mu2c - muc * muc + 1e-5)
        af = (p['gh'][:_HA] * invf)[None]
        cf = (p['bh'][:_HA] - muf * p['gh'][:_HA] * invf)[None]
        ac = (p['gh'][_HA:] * invc)[None]
        cc = (p['bh'][_HA:] - muc * p['gh'][_HA:] * invc)[None]

        summed, stats2 = pl.pallas_call(
            _pass2_body,
            grid=grid,
            in_specs=[dist_spec, gath_spec, atom_spec, full(filt_arr.shape)] + win +
                     [full((1, _HA)), full((1, _HA)), full((1, _HA)), full((1, _HA))],
            out_specs=[atom_spec, full((2, _HA))],
            out_shape=[jax.ShapeDtypeStruct((B, N, _HA), jnp.float32),
                       jax.ShapeDtypeStruct((2, _HA), jnp.float32)],
        )(dist_c, gath, emb, filt_arr, w1f, w1c, w2f, w2c, w3f, w3c,
          bf, bc_, af, cf, ac, cc)

        cnt2 = float(B * N)
        mu2 = stats2[0] / cnt2
        var2 = stats2[1] / cnt2 - mu2 * mu2
        inv2 = jax.lax.rsqrt(var2 + 1e-5)
        a2 = (p['go'] * inv2)[None, None]
        c2 = (p['bo'] - mu2 * p['go'] * inv2)[None, None]

        emb = pl.pallas_call(
            _pass3_body,
            grid=(B,),
            in_specs=[pl.BlockSpec((1, N, _HA), lambda b: (b, 0, 0)),
                      pl.BlockSpec((1, N, _HA), lambda b: (b, 0, 0)),
                      pl.BlockSpec((1, 1, _HA), lambda b: (0, 0, 0)),
                      pl.BlockSpec((1, 1, _HA), lambda b: (0, 0, 0))],
            out_specs=pl.BlockSpec((1, N, _HA), lambda b: (b, 0, 0)),
            out_shape=jax.ShapeDtypeStruct((B, N, _HA), jnp.float32),
        )(summed, emb, a2, c2)

    out = pl.pallas_call(
        _head_body,
        in_specs=[pl.BlockSpec(emb.shape, lambda: (0, 0, 0)),
                  pl.BlockSpec(Wc.shape, lambda: (0, 0)),
                  pl.BlockSpec((1, Wc.shape[1]), lambda: (0, 0))],
        out_specs=pl.BlockSpec((B, Wc.shape[1]), lambda: (0, 0)),
        out_shape=jax.ShapeDtypeStruct((B, Wc.shape[1]), jnp.float32),
    )(emb, Wc, bc[None])
    return out
